# Initial kernel scaffold; baseline (speedup 1.0000x reference)
import jax
import jax.numpy as jnp
from jax.experimental import pallas as pl


def kernel(x, edge_index_0, edge_index_1, edge_index_2, edge_index_3, weight, h_bias):
    raise NotImplementedError("write your pallas kernel here")


# speedup vs baseline: 4.9325x; 4.9325x over previous
import functools

import jax
import jax.numpy as jnp
from jax import lax
from jax.experimental import pallas as pl
from jax.experimental.pallas import tpu as pltpu
from jax.experimental.pallas import tpu_sc as plsc

N = 10000
R = 4
E = 80000
D = 128
K = 128
SLOTS = E // K
NS = 16
NC = 2
ITERS = (SLOTS + NS - 1) // NS
DEGW = 16
CH = 40
MAXCH = 16


def _sc_body(x_hbm, ei_hbm, agg_out, deg_out,
             srcb, dstb, rows, ones, zrow, zdeg, aggb, degb,
             agg_sh, deg_sh, sem):
    c = lax.axis_index("c")
    s = lax.axis_index("s")

    def fill_ones(i, carry):
        ones[i] = jnp.full((16,), 1.0, jnp.float32)
        return carry
    lax.fori_loop(0, K, fill_ones, 0)

    def fill_zrow(i, carry):
        for j in range(D // 16):
            zrow[i, pl.ds(j * 16, 16)] = jnp.zeros((16,), jnp.float32)
        return carry
    lax.fori_loop(0, CH, fill_zrow, 0)

    def fill_zdeg(i, carry):
        zdeg[i] = jnp.zeros((16,), jnp.float32)
        return carry
    lax.fori_loop(0, CH, fill_zdeg, 0)

    row0 = jnp.where(s < 10, 640 * s, 6400 + 600 * (s - 10))
    nch = jnp.where(s < 10, 16, 15)

    for j in range(R // NC):
        r = c * (R // NC) + j

        def zero_chunk(z, carry):
            @pl.when(z < nch)
            def _():
                sl = pl.ds(row0 + z * CH, CH)
                pltpu.sync_copy(zrow, agg_sh.at[sl, :])
                pltpu.sync_copy(zdeg, deg_sh.at[sl, :])
            return carry
        lax.fori_loop(0, MAXCH, zero_chunk, 0)
        plsc.subcore_barrier()

        def slot_body(i, carry):
            slot = s + i * NS

            @pl.when(slot < SLOTS)
            def _():
                off = slot * K
                base = r * 2 * E
                pltpu.sync_copy(ei_hbm.at[pl.ds(base + off, K)], srcb)
                pltpu.sync_copy(ei_hbm.at[pl.ds(base + E + off, K)], dstb)
                pltpu.async_copy(x_hbm.at[srcb], rows, sem).wait()
                pltpu.sync_copy(rows, agg_sh.at[dstb], add=True)
                pltpu.sync_copy(ones, deg_sh.at[dstb], add=True)

            return carry
        lax.fori_loop(0, ITERS, slot_body, 0)
        plsc.subcore_barrier()

        def out_chunk(z, carry):
            @pl.when(z < nch)
            def _():
                sl = pl.ds(row0 + z * CH, CH)
                pltpu.sync_copy(agg_sh.at[sl, :], aggb)
                pltpu.sync_copy(aggb, agg_out.at[r, sl, :])
                pltpu.sync_copy(deg_sh.at[sl, :], degb)
                pltpu.sync_copy(degb, deg_out.at[r, sl, :])
            return carry
        lax.fori_loop(0, MAXCH, out_chunk, 0)
        plsc.subcore_barrier()


_sc_aggregate = functools.partial(
    pl.kernel,
    out_type=[
        jax.ShapeDtypeStruct((R, N, D), jnp.float32),
        jax.ShapeDtypeStruct((R, N, DEGW), jnp.float32),
    ],
    mesh=plsc.VectorSubcoreMesh(core_axis_name="c", subcore_axis_name="s"),
    compiler_params=pltpu.CompilerParams(use_tc_tiling_on_sc=False),
    scratch_types=[
        pltpu.VMEM((K,), jnp.int32),
        pltpu.VMEM((K,), jnp.int32),
        pltpu.VMEM((K, D), jnp.float32),
        pltpu.VMEM((K, DEGW), jnp.float32),
        pltpu.VMEM((CH, D), jnp.float32),
        pltpu.VMEM((CH, DEGW), jnp.float32),
        pltpu.VMEM((CH, D), jnp.float32),
        pltpu.VMEM((CH, DEGW), jnp.float32),
        pltpu.VMEM_SHARED((N, D), jnp.float32),
        pltpu.VMEM_SHARED((N, DEGW), jnp.float32),
        pltpu.SemaphoreType.DMA,
    ],
)(_sc_body)


BN = 1000


def _tc_body(agg_ref, deg_ref, w_ref, b_ref, o_ref):
    acc = jnp.zeros((BN, D), jnp.float32)
    for r in range(R):
        deg = deg_ref[r, :, 0:1]
        norm = 1.0 / jnp.maximum(deg, 1.0)
        acc = acc + jnp.dot(agg_ref[r] * norm, w_ref[r],
                            preferred_element_type=jnp.float32)
    o_ref[...] = acc + b_ref[...]


def _tc_combine(agg, deg, weight, bias):
    return pl.pallas_call(
        _tc_body,
        grid=(N // BN,),
        in_specs=[
            pl.BlockSpec((R, BN, D), lambda i: (0, i, 0)),
            pl.BlockSpec((R, BN, DEGW), lambda i: (0, i, 0)),
            pl.BlockSpec((R, D, D), lambda i: (0, 0, 0)),
            pl.BlockSpec((1, D), lambda i: (0, 0)),
        ],
        out_specs=pl.BlockSpec((BN, D), lambda i: (i, 0)),
        out_shape=jax.ShapeDtypeStruct((N, D), jnp.float32),
    )(agg, deg, weight, bias)


def kernel(x, edge_index_0, edge_index_1, edge_index_2, edge_index_3,
           weight, h_bias):
    ei = jnp.stack([edge_index_0, edge_index_1, edge_index_2,
                    edge_index_3]).reshape(-1)
    agg, deg = _sc_aggregate(x, ei)
    return _tc_combine(agg, deg, weight, h_bias.reshape(1, D))
